# baseline (speedup 1.0000x reference)
import functools

import jax
import jax.numpy as jnp
import numpy as np
from jax import lax
from jax.experimental import pallas as pl
from jax.experimental.pallas import tpu as pltpu
from jax.experimental.pallas import tpu_sc as plsc

N = 10000
E = 320000
G = 64
D_IN = 128
PE = 16
H = 128
HEADS = 4
DH = H // HEADS
LAYERS = 4
LN_EPS = 1e-5

T = 128
KT = 256
NP = 10240
NT = NP // T
NT2 = NP // KT
GP = 128
PADG = G

NC = 2
NS = 16
EB = 128
EPB = 80
NEB = NC * NS * EPB
EPAD = NEB * EB - E
RPT = NP // NS

_SQ2I = np.float32(1.0 / np.sqrt(2.0))


def _mm(a, b):
    return lax.dot_general(a, b, (((1,), (0,)), ((), ())),
                           preferred_element_type=jnp.float32)


def _mmt(a, b):
    return lax.dot_general(a, b, (((1,), (1,)), ((), ())),
                           preferred_element_type=jnp.float32)


def _sc_agg(h, src2d, dst2d, zeros):
    mesh = plsc.VectorSubcoreMesh(core_axis_name="c", subcore_axis_name="s")

    @functools.partial(
        pl.kernel,
        out_type=jax.ShapeDtypeStruct((NC * NP, H), jnp.float32),
        mesh=mesh,
        scratch_types=[
            pltpu.VMEM((EPB * EB,), jnp.int32),
            pltpu.VMEM((EB,), jnp.int32),
            pltpu.VMEM((EB,), jnp.int32),
            pltpu.VMEM((EB, H), jnp.float32),
            pltpu.VMEM((EB, H), jnp.float32),
            pltpu.VMEM_SHARED((NP, H), jnp.float32),
            pltpu.SemaphoreType.DMA,
            pltpu.SemaphoreType.DMA,
            pltpu.SemaphoreType.DMA,
            pltpu.SemaphoreType.DMA,
        ],
    )
    def k(h_hbm, src_hbm, dst_hbm, z_hbm, out_hbm,
          sall, d0, d1, r0, r1, acc, semg0, semg1, semd0, semd1):
        c = lax.axis_index("c")
        s = lax.axis_index("s")
        base_e = (c * NS + s) * (EPB * EB)
        pltpu.sync_copy(src_hbm.at[pl.ds(base_e, EPB * EB)], sall)
        pltpu.sync_copy(z_hbm.at[pl.ds(s * RPT, RPT)],
                        acc.at[pl.ds(s * RPT, RPT)])
        plsc.subcore_barrier()

        def gath(j, r, sem):
            pltpu.async_copy(h_hbm.at[sall.at[pl.ds(j * EB, EB)]], r, sem)

        def gath_wait(j, r, sem):
            pltpu.make_async_copy(
                h_hbm.at[sall.at[pl.ds(j * EB, EB)]], r, sem).wait()

        def dcopy(j, d, sem):
            pltpu.async_copy(dst_hbm.at[pl.ds(base_e + j * EB, EB)], d, sem)

        def dcopy_wait(j, d, sem):
            pltpu.make_async_copy(
                dst_hbm.at[pl.ds(base_e + j * EB, EB)], d, sem).wait()

        gath(0, r0, semg0)
        dcopy(0, d0, semd0)
        gath(1, r1, semg1)
        dcopy(1, d1, semd1)

        def pair(kk, carry):
            j0 = 2 * kk
            gath_wait(j0, r0, semg0)
            dcopy_wait(j0, d0, semd0)
            pltpu.sync_copy(r0, acc.at[d0], add=True)
            gath(j0 + 2, r0, semg0)
            dcopy(j0 + 2, d0, semd0)
            gath_wait(j0 + 1, r1, semg1)
            dcopy_wait(j0 + 1, d1, semd1)
            pltpu.sync_copy(r1, acc.at[d1], add=True)
            gath(j0 + 3, r1, semg1)
            dcopy(j0 + 3, d1, semd1)
            return carry

        lax.fori_loop(0, EPB // 2 - 1, pair, 0)
        j0 = EPB - 2
        gath_wait(j0, r0, semg0)
        dcopy_wait(j0, d0, semd0)
        pltpu.sync_copy(r0, acc.at[d0], add=True)
        gath_wait(j0 + 1, r1, semg1)
        dcopy_wait(j0 + 1, d1, semd1)
        pltpu.sync_copy(r1, acc.at[d1], add=True)
        plsc.subcore_barrier()
        pltpu.sync_copy(acc.at[pl.ds(s * RPT, RPT)],
                        out_hbm.at[pl.ds(c * NP + s * RPT, RPT)])

    return k(h, src2d.reshape(-1), dst2d.reshape(-1), zeros)


def _h0qkv_body(xc_ref, w1_ref, b1_ref, w2_ref, b2_ref,
                wq_ref, bq_ref, wk_ref, bk_ref, wv_ref, bv_ref,
                o_ref, q_ref, k_ref, v_ref):
    z = jnp.maximum(_mm(xc_ref[...], w1_ref[...]) + b1_ref[...], 0.0)
    h = _mm(z, w2_ref[...]) + b2_ref[...]
    o_ref[...] = h
    q_ref[...] = _mm(h, wq_ref[...]) + bq_ref[...]
    k_ref[...] = _mm(h, wk_ref[...]) + bk_ref[...]
    v_ref[...] = _mm(h, wv_ref[...]) + bv_ref[...]


def _stat_rows(t):
    ones_h = jnp.ones((H, 1), jnp.float32)
    return jnp.concatenate(
        [_mm(t, ones_h), _mm(t * t, ones_h),
         jnp.zeros((T, 6), jnp.float32)], axis=1)


def _accum(i, sg_ref, ps):
    @pl.when(i == 0)
    def _():
        sg_ref[...] = ps

    @pl.when(i > 0)
    def _():
        sg_ref[...] += ps


def _attn_body(q_ref, k_ref, v_ref, bqc_ref, bt_ref, lo_ref, hi_ref,
               wo_ref, bo_ref, h_ref, m_ref, o_ref, sg_ref):
    i = pl.program_id(0)
    lo = lo_ref[0, i]
    hi = hi_ref[0, i]
    q = q_ref[...]
    bq = bqc_ref[...]
    scale = np.float32(1.0 / np.sqrt(DH))
    qh = [q[:, j * DH:(j + 1) * DH] * scale for j in range(HEADS)]

    ones_col = jnp.ones((KT, 1), jnp.float32)

    def body(kt, carry):
        m, l, acc = carry
        kk = k_ref[pl.ds(kt * KT, KT), :]
        vv = v_ref[pl.ds(kt * KT, KT), :]
        bk = bt_ref[pl.ds(kt, 1), :]
        maskf = (bq == bk).astype(jnp.float32)
        ss = [_mmt(qh[j], kk[:, j * DH:(j + 1) * DH]) for j in range(HEADS)]
        smax = jnp.maximum(jnp.maximum(ss[0], ss[1]),
                           jnp.maximum(ss[2], ss[3]))
        smax = smax * maskf - 1e9 * (1.0 - maskf)
        m_new = jnp.maximum(m, jnp.max(smax, 1, keepdims=True))
        alpha = jnp.exp(m - m_new)
        nls, naccs = [], []
        for j in range(HEADS):
            p = jnp.exp(jnp.minimum(ss[j] - m_new, 0.0)) * maskf
            nls.append(l[:, j:j + 1] * alpha + _mm(p, ones_col))
            naccs.append(acc[:, j * DH:(j + 1) * DH] * alpha
                         + _mm(p, vv[:, j * DH:(j + 1) * DH]))
        return m_new, jnp.concatenate(nls, 1), jnp.concatenate(naccs, 1)

    m0 = jnp.full((T, 1), -1e9, jnp.float32)
    l0 = jnp.zeros((T, HEADS), jnp.float32)
    a0 = jnp.zeros((T, H), jnp.float32)
    m, l, acc = lax.fori_loop(lo, hi, body, (m0, l0, a0))
    o = jnp.concatenate(
        [acc[:, j * DH:(j + 1) * DH] / l[:, j:j + 1] for j in range(HEADS)],
        axis=1)
    ha = _mm(o, wo_ref[...]) + bo_ref[...] + h_ref[...]
    o_ref[...] = ha
    _accum(i, sg_ref, _mm(m_ref[...], _stat_rows(ha)))


def _gin_body(h_ref, a0_ref, a1_ref, eps_ref, w1_ref, b1_ref, w2_ref, b2_ref,
              m_ref, o_ref, sg_ref):
    i = pl.program_id(0)
    h = h_ref[...]
    z = h + eps_ref[...] * h + a0_ref[...] + a1_ref[...]
    z1 = jnp.maximum(_mm(z, w1_ref[...]) + b1_ref[...], 0.0)
    hl = _mm(z1, w2_ref[...]) + b2_ref[...] + h
    o_ref[...] = hl
    _accum(i, sg_ref, _mm(m_ref[...], _stat_rows(hl)))


def _gstats(sg, nv):
    mean = sg[:, 0:1] / nv
    var = sg[:, 1:2] / nv - mean * mean
    return mean, lax.rsqrt(var + LN_EPS)


def _mid_body(hl_ref, ha_ref, sgl_ref, sga_ref, nv_ref, mt_ref,
              g1_ref, c1_ref, g2_ref, c2_ref,
              w1_ref, bf1_ref, w2_ref, bf2_ref, m_ref, o_ref, sg_ref):
    i = pl.program_id(0)
    nv = nv_ref[...]
    m1, i1 = _gstats(sgl_ref[...], nv)
    m2, i2 = _gstats(sga_ref[...], nv)
    st = jnp.concatenate([m1, i1, m2, i2, jnp.zeros((GP, 4), jnp.float32)], 1)
    ns = _mm(mt_ref[...], st)
    hl = (hl_ref[...] - ns[:, 0:1]) * ns[:, 1:2] * g1_ref[...] + c1_ref[...]
    ha = (ha_ref[...] - ns[:, 2:3]) * ns[:, 3:4] * g2_ref[...] + c2_ref[...]
    op = hl + ha
    z = _mm(op, w1_ref[...]) + bf1_ref[...]
    ge = 0.5 * z * (1.0 + lax.erf(z * _SQ2I))
    out2 = _mm(ge, w2_ref[...]) + bf2_ref[...] + op
    o_ref[...] = out2
    _accum(i, sg_ref, _mm(m_ref[...], _stat_rows(out2)))


def _ln3_node(h_ref, o2_ref, sg_ref, nv_ref, mt_ref, g3_ref, c3_ref):
    m3, i3 = _gstats(sg_ref[...], nv_ref[...])
    st = jnp.concatenate([m3, i3, jnp.zeros((GP, 6), jnp.float32)], 1)
    ns = _mm(mt_ref[...], st)
    return (h_ref[...]
            + (o2_ref[...] - ns[:, 0:1]) * ns[:, 1:2] * g3_ref[...]
            + c3_ref[...])


def _finqkv_body(h_ref, o2_ref, sg_ref, nv_ref, mt_ref, g3_ref, c3_ref,
                 wq_ref, bq_ref, wk_ref, bk_ref, wv_ref, bv_ref,
                 o_ref, q_ref, k_ref, v_ref):
    hn = _ln3_node(h_ref, o2_ref, sg_ref, nv_ref, mt_ref, g3_ref, c3_ref)
    o_ref[...] = hn
    q_ref[...] = _mm(hn, wq_ref[...]) + bq_ref[...]
    k_ref[...] = _mm(hn, wk_ref[...]) + bk_ref[...]
    v_ref[...] = _mm(hn, wv_ref[...]) + bv_ref[...]


def _finpost_body(h_ref, o2_ref, sg_ref, nv_ref, mt_ref, g3_ref, c3_ref,
                  w1_ref, b1_ref, w2_ref, b2_ref, m_ref, o_ref, sgp_ref):
    i = pl.program_id(0)
    hn = _ln3_node(h_ref, o2_ref, sg_ref, nv_ref, mt_ref, g3_ref, c3_ref)
    o_ref[...] = hn
    z = jnp.maximum(_mm(hn, w1_ref[...]) + b1_ref[...], 0.0)
    hp = hn + _mm(z, w2_ref[...]) + b2_ref[...]
    _accum(i, sgp_ref, _mm(m_ref[...], hp))


def _read_body(sg_ref, dn_ref, w1_ref, b1_ref, w2_ref, b2_ref, o_ref):
    g = sg_ref[...] / dn_ref[...]
    z = jnp.maximum(_mm(g, w1_ref[...]) + b1_ref[...], 0.0)
    o_ref[...] = _mm(z, w2_ref[...]) + b2_ref[...]


_FULL2 = lambda i: (0, 0)
_TILE = lambda i: (i, 0)


def _spec_full(shape):
    return pl.BlockSpec(shape, _FULL2)


def _spec_tile(cols):
    return pl.BlockSpec((T, cols), _TILE)


def kernel(x, edge_index, batch, lap_pe, params):
    batch = batch.astype(jnp.int32)
    epad = jnp.arange(EPAD, dtype=jnp.int32)
    src2d = jnp.concatenate(
        [edge_index[0].astype(jnp.int32), (epad * 997) % N]).reshape(NEB, EB)
    dst2d = jnp.concatenate(
        [edge_index[1].astype(jnp.int32), N + epad % (NP - N)]).reshape(NEB, EB)

    counts = jnp.bincount(batch, length=G)
    starts = jnp.cumsum(counts) - counts
    batch_pad = jnp.concatenate(
        [batch, jnp.full((NP - N,), PADG, jnp.int32)])
    counts_p = jnp.concatenate(
        [counts, jnp.array([NP - N], counts.dtype),
         jnp.zeros((GP - G - 1,), counts.dtype)])
    starts_p = jnp.concatenate(
        [starts, jnp.array([N], starts.dtype),
         jnp.full((GP - G - 1,), NP, starts.dtype)])
    ends_p = starts_p + counts_p
    gids = jnp.arange(GP, dtype=jnp.int32)
    mmat = (batch_pad[None, :] == gids[:, None]).astype(jnp.float32)
    mtmat = (batch_pad[:, None] == gids[None, :]).astype(jnp.float32)
    bq_col = batch_pad.reshape(NP, 1)
    bt = batch_pad.reshape(NT2, KT)
    first_g = batch_pad[0::T]
    last_g = batch_pad[T - 1::T]
    kv_lo = (starts_p[first_g] // KT).astype(jnp.int32).reshape(1, NT)
    kv_hi = ((ends_p[last_g] + (KT - 1)) // KT).astype(jnp.int32).reshape(1, NT)
    normv = jnp.maximum(counts_p.astype(jnp.float32) * H, 1.0).reshape(GP, 1)
    denom = jnp.maximum(counts_p.astype(jnp.float32), 1.0).reshape(GP, 1)
    zeros_np = jnp.zeros((NP, H), jnp.float32)
    xc = jnp.concatenate([x, lap_pe], axis=1)
    xc = jnp.concatenate(
        [xc, jnp.zeros((NP - N, D_IN + PE), jnp.float32)], axis=0)

    def lin(p):
        return p["w"], p["b"].reshape(1, -1)

    smem_spec = pl.BlockSpec(memory_space=pltpu.SMEM)

    f32 = jnp.float32

    def attnw(lp):
        return lin(lp["attn"]["q"]) + lin(lp["attn"]["k"]) + lin(lp["attn"]["v"])

    nm = params["node_mlp"]
    w1, b1 = lin(nm["l1"])
    w2, b2 = lin(nm["l2"])
    h, q, k, v = pl.pallas_call(
        _h0qkv_body,
        grid=(NT,),
        in_specs=[_spec_tile(D_IN + PE), _spec_full((D_IN + PE, H)),
                  _spec_full((1, H)), _spec_full((H, H)), _spec_full((1, H))]
                 + [_spec_full((H, H)), _spec_full((1, H))] * 3,
        out_specs=[_spec_tile(H)] * 4,
        out_shape=[jax.ShapeDtypeStruct((NP, H), f32)] * 4,
    )(xc, w1, b1, w2, b2, *attnw(params["layers"][0]))

    for li, lp in enumerate(params["layers"]):
        agg2 = _sc_agg(h, src2d, dst2d, zeros_np)

        wo, bo = lin(lp["attn"]["o"])
        ha_pre, sg_ha = pl.pallas_call(
            _attn_body,
            grid=(NT,),
            in_specs=[_spec_tile(H), _spec_full((NP, H)), _spec_full((NP, H)),
                      pl.BlockSpec((T, 1), _TILE), _spec_full((NT2, KT)),
                      smem_spec, smem_spec,
                      _spec_full((H, H)), _spec_full((1, H)), _spec_tile(H),
                      pl.BlockSpec((GP, T), lambda i: (0, i))],
            out_specs=[_spec_tile(H), pl.BlockSpec((GP, 8), _FULL2)],
            out_shape=[jax.ShapeDtypeStruct((NP, H), f32),
                       jax.ShapeDtypeStruct((GP, 8), f32)],
        )(q, k, v, bq_col, bt, kv_lo, kv_hi, wo, bo, h, mmat)

        gw1, gb1 = lin(lp["gin_mlp"]["l1"])
        gw2, gb2 = lin(lp["gin_mlp"]["l2"])
        geps = lp["gin_eps"].reshape(1, 1)
        hl_pre, sg_hl = pl.pallas_call(
            _gin_body,
            grid=(NT,),
            in_specs=[_spec_tile(H),
                      pl.BlockSpec((T, H), _TILE),
                      pl.BlockSpec((T, H), lambda i: (i + NT, 0)),
                      _spec_full((1, 1)),
                      _spec_full((H, H)), _spec_full((1, H)),
                      _spec_full((H, H)), _spec_full((1, H)),
                      pl.BlockSpec((GP, T), lambda i: (0, i))],
            out_specs=[_spec_tile(H), pl.BlockSpec((GP, 8), _FULL2)],
            out_shape=[jax.ShapeDtypeStruct((NP, H), f32),
                       jax.ShapeDtypeStruct((GP, 8), f32)],
        )(h, agg2, agg2, geps, gw1, gb1, gw2, gb2, mmat)

        g1 = lp["norm1"]["g"].reshape(1, H)
        c1 = lp["norm1"]["b"].reshape(1, H)
        g2 = lp["norm2"]["g"].reshape(1, H)
        c2 = lp["norm2"]["b"].reshape(1, H)
        fw1, fb1 = lin(lp["ff"]["l1"])
        fw2, fb2 = lin(lp["ff"]["l2"])
        out2, sg_o = pl.pallas_call(
            _mid_body,
            grid=(NT,),
            in_specs=[_spec_tile(H), _spec_tile(H), _spec_full((GP, 8)),
                      _spec_full((GP, 8)), _spec_full((GP, 1)),
                      _spec_tile(GP),
                      _spec_full((1, H)), _spec_full((1, H)),
                      _spec_full((1, H)), _spec_full((1, H)),
                      _spec_full((H, 2 * H)), _spec_full((1, 2 * H)),
                      _spec_full((2 * H, H)), _spec_full((1, H)),
                      pl.BlockSpec((GP, T), lambda i: (0, i))],
            out_specs=[_spec_tile(H), pl.BlockSpec((GP, 8), _FULL2)],
            out_shape=[jax.ShapeDtypeStruct((NP, H), f32),
                       jax.ShapeDtypeStruct((GP, 8), f32)],
        )(hl_pre, ha_pre, sg_hl, sg_ha, normv, mtmat, g1, c1, g2, c2,
          fw1, fb1, fw2, fb2, mmat)

        g3 = lp["norm3"]["g"].reshape(1, H)
        c3 = lp["norm3"]["b"].reshape(1, H)
        if li < LAYERS - 1:
            h, q, k, v = pl.pallas_call(
                _finqkv_body,
                grid=(NT,),
                in_specs=[_spec_tile(H), _spec_tile(H), _spec_full((GP, 8)),
                          _spec_full((GP, 1)), _spec_tile(GP),
                          _spec_full((1, H)), _spec_full((1, H))]
                         + [_spec_full((H, H)), _spec_full((1, H))] * 3,
                out_specs=[_spec_tile(H)] * 4,
                out_shape=[jax.ShapeDtypeStruct((NP, H), f32)] * 4,
            )(h, out2, sg_o, normv, mtmat, g3, c3,
              *attnw(params["layers"][li + 1]))
        else:
            pw1, pb1 = lin(params["postnet"]["l1"])
            pw2, pb2 = lin(params["postnet"]["l2"])
            _, sg_pool = pl.pallas_call(
                _finpost_body,
                grid=(NT,),
                in_specs=[_spec_tile(H), _spec_tile(H), _spec_full((GP, 8)),
                          _spec_full((GP, 1)), _spec_tile(GP),
                          _spec_full((1, H)), _spec_full((1, H)),
                          _spec_full((H, H)), _spec_full((1, H)),
                          _spec_full((H, H)), _spec_full((1, H)),
                          pl.BlockSpec((GP, T), lambda i: (0, i))],
                out_specs=[_spec_tile(H), pl.BlockSpec((GP, H), _FULL2)],
                out_shape=[jax.ShapeDtypeStruct((NP, H), f32),
                           jax.ShapeDtypeStruct((GP, H), f32)],
            )(h, out2, sg_o, normv, mtmat, g3, c3,
              pw1, pb1, pw2, pb2, mmat)

    rw1, rb1 = lin(params["readout"]["l1"])
    rw2, rb2 = lin(params["readout"]["l2"])
    out = pl.pallas_call(
        _read_body,
        grid=(1,),
        in_specs=[_spec_full((GP, H)), _spec_full((GP, 1)),
                  _spec_full((H, H)), _spec_full((1, H)),
                  _spec_full((H, 1)), _spec_full((1, 1))],
        out_specs=_spec_full((GP, 1)),
        out_shape=jax.ShapeDtypeStruct((GP, 1), jnp.float32),
    )(sg_pool, denom, rw1, rb1, rw2, rb2)
    return out[:G]


# speedup vs baseline: 1.7439x; 1.7439x over previous
import functools

import jax
import jax.numpy as jnp
import numpy as np
from jax import lax
from jax.experimental import pallas as pl
from jax.experimental.pallas import tpu as pltpu
from jax.experimental.pallas import tpu_sc as plsc

N = 10000
E = 320000
G = 64
D_IN = 128
PE = 16
H = 128
HEADS = 4
DH = H // HEADS
LAYERS = 4
LN_EPS = 1e-5

T = 128
KT = 256
TN = 256
NP = 10240
NT = NP // T
NT2 = NP // KT
NTN = NP // TN
GP = 128
PADG = G

NC = 2
NS = 16
EB = 128
EPB = 80
NEB = NC * NS * EPB
EPAD = NEB * EB - E
RPT = NP // NS

_SQ2I = np.float32(1.0 / np.sqrt(2.0))


def _mm(a, b):
    return lax.dot_general(a, b, (((1,), (0,)), ((), ())),
                           preferred_element_type=jnp.float32)


def _mmt(a, b):
    return lax.dot_general(a, b, (((1,), (1,)), ((), ())),
                           preferred_element_type=jnp.float32)


def _sc_agg(h, src2d, dst2d, zeros):
    mesh = plsc.VectorSubcoreMesh(core_axis_name="c", subcore_axis_name="s")

    @functools.partial(
        pl.kernel,
        out_type=jax.ShapeDtypeStruct((NC * NP, H), jnp.float32),
        mesh=mesh,
        scratch_types=[
            pltpu.VMEM((EPB * EB,), jnp.int32),
            pltpu.VMEM((EB,), jnp.int32),
            pltpu.VMEM((EB,), jnp.int32),
            pltpu.VMEM((EB, H), jnp.float32),
            pltpu.VMEM((EB, H), jnp.float32),
            pltpu.VMEM_SHARED((NP, H), jnp.float32),
            pltpu.SemaphoreType.DMA,
            pltpu.SemaphoreType.DMA,
            pltpu.SemaphoreType.DMA,
            pltpu.SemaphoreType.DMA,
        ],
    )
    def k(h_hbm, src_hbm, dst_hbm, z_hbm, out_hbm,
          sall, d0, d1, r0, r1, acc, semg0, semg1, semd0, semd1):
        c = lax.axis_index("c")
        s = lax.axis_index("s")
        base_e = (c * NS + s) * (EPB * EB)
        pltpu.sync_copy(src_hbm.at[pl.ds(base_e, EPB * EB)], sall)
        pltpu.sync_copy(z_hbm.at[pl.ds(s * RPT, RPT)],
                        acc.at[pl.ds(s * RPT, RPT)])
        plsc.subcore_barrier()

        def gath(j, r, sem):
            pltpu.async_copy(h_hbm.at[sall.at[pl.ds(j * EB, EB)]], r, sem)

        def gath_wait(j, r, sem):
            pltpu.make_async_copy(
                h_hbm.at[sall.at[pl.ds(j * EB, EB)]], r, sem).wait()

        def dcopy(j, d, sem):
            pltpu.async_copy(dst_hbm.at[pl.ds(base_e + j * EB, EB)], d, sem)

        def dcopy_wait(j, d, sem):
            pltpu.make_async_copy(
                dst_hbm.at[pl.ds(base_e + j * EB, EB)], d, sem).wait()

        gath(0, r0, semg0)
        dcopy(0, d0, semd0)
        gath(1, r1, semg1)
        dcopy(1, d1, semd1)

        def pair(kk, carry):
            j0 = 2 * kk
            gath_wait(j0, r0, semg0)
            dcopy_wait(j0, d0, semd0)
            pltpu.sync_copy(r0, acc.at[d0], add=True)
            gath(j0 + 2, r0, semg0)
            dcopy(j0 + 2, d0, semd0)
            gath_wait(j0 + 1, r1, semg1)
            dcopy_wait(j0 + 1, d1, semd1)
            pltpu.sync_copy(r1, acc.at[d1], add=True)
            gath(j0 + 3, r1, semg1)
            dcopy(j0 + 3, d1, semd1)
            return carry

        lax.fori_loop(0, EPB // 2 - 1, pair, 0)
        j0 = EPB - 2
        gath_wait(j0, r0, semg0)
        dcopy_wait(j0, d0, semd0)
        pltpu.sync_copy(r0, acc.at[d0], add=True)
        gath_wait(j0 + 1, r1, semg1)
        dcopy_wait(j0 + 1, d1, semd1)
        pltpu.sync_copy(r1, acc.at[d1], add=True)
        plsc.subcore_barrier()
        pltpu.sync_copy(acc.at[pl.ds(s * RPT, RPT)],
                        out_hbm.at[pl.ds(c * NP + s * RPT, RPT)])

    return k(h, src2d.reshape(-1), dst2d.reshape(-1), zeros)


def _h0qkv_body(xc_ref, w1_ref, b1_ref, w2_ref, b2_ref,
                wq_ref, bq_ref, wk_ref, bk_ref, wv_ref, bv_ref,
                o_ref, q_ref, k_ref, v_ref):
    z = jnp.maximum(_mm(xc_ref[...], w1_ref[...]) + b1_ref[...], 0.0)
    h = _mm(z, w2_ref[...]) + b2_ref[...]
    o_ref[...] = h
    q_ref[...] = _mm(h, wq_ref[...]) + bq_ref[...]
    k_ref[...] = _mm(h, wk_ref[...]) + bk_ref[...]
    v_ref[...] = _mm(h, wv_ref[...]) + bv_ref[...]


def _stat_rows(t):
    return jnp.concatenate(
        [jnp.sum(t, 1, keepdims=True), jnp.sum(t * t, 1, keepdims=True),
         jnp.zeros((t.shape[0], 6), jnp.float32)], axis=1)


def _accum(i, sg_ref, ps):
    @pl.when(i == 0)
    def _():
        sg_ref[...] = ps

    @pl.when(i > 0)
    def _():
        sg_ref[...] += ps


def _attn_body(q_ref, k_ref, v_ref, bqc_ref, bt_ref, lo_ref, hi_ref,
               wo_ref, bo_ref, h_ref, m_ref, o_ref, sg_ref):
    i = pl.program_id(0)
    lo = lo_ref[0, i]
    hi = hi_ref[0, i]
    q = q_ref[...]
    bq = bqc_ref[...]
    scale = np.float32(1.0 / np.sqrt(DH))
    qh = [q[:, j * DH:(j + 1) * DH] * scale for j in range(HEADS)]

    ones_col = jnp.ones((KT, 1), jnp.float32)

    def body(kt, carry):
        m, l, acc = carry
        kk = k_ref[pl.ds(kt * KT, KT), :]
        vv = v_ref[pl.ds(kt * KT, KT), :]
        bk = bt_ref[pl.ds(kt, 1), :]
        maskf = (bq == bk).astype(jnp.float32)
        ss = [_mmt(qh[j], kk[:, j * DH:(j + 1) * DH]) for j in range(HEADS)]
        smax = jnp.maximum(jnp.maximum(ss[0], ss[1]),
                           jnp.maximum(ss[2], ss[3]))
        smax = smax * maskf - 1e9 * (1.0 - maskf)
        m_new = jnp.maximum(m, jnp.max(smax, 1, keepdims=True))
        alpha = jnp.exp(m - m_new)
        nls, naccs = [], []
        for j in range(HEADS):
            p = jnp.exp(jnp.minimum(ss[j] - m_new, 0.0)) * maskf
            nls.append(l[:, j:j + 1] * alpha + _mm(p, ones_col))
            naccs.append(acc[:, j * DH:(j + 1) * DH] * alpha
                         + _mm(p, vv[:, j * DH:(j + 1) * DH]))
        return m_new, jnp.concatenate(nls, 1), jnp.concatenate(naccs, 1)

    m0 = jnp.full((T, 1), -1e9, jnp.float32)
    l0 = jnp.zeros((T, HEADS), jnp.float32)
    a0 = jnp.zeros((T, H), jnp.float32)
    m, l, acc = lax.fori_loop(lo, hi, body, (m0, l0, a0))
    o = jnp.concatenate(
        [acc[:, j * DH:(j + 1) * DH] / l[:, j:j + 1] for j in range(HEADS)],
        axis=1)
    ha = _mm(o, wo_ref[...]) + bo_ref[...] + h_ref[...]
    o_ref[...] = ha
    _accum(i, sg_ref, _mm(m_ref[...], _stat_rows(ha)))


def _gin_body(h_ref, a0_ref, a1_ref, eps_ref, w1_ref, b1_ref, w2_ref, b2_ref,
              m_ref, o_ref, sg_ref):
    i = pl.program_id(0)
    h = h_ref[...]
    z = h + eps_ref[...] * h + a0_ref[...] + a1_ref[...]
    z1 = jnp.maximum(_mm(z, w1_ref[...]) + b1_ref[...], 0.0)
    hl = _mm(z1, w2_ref[...]) + b2_ref[...] + h
    o_ref[...] = hl
    _accum(i, sg_ref, _mm(m_ref[...], _stat_rows(hl)))


def _gstats(sg, nv):
    mean = sg[:, 0:1] / nv
    var = sg[:, 1:2] / nv - mean * mean
    return mean, lax.rsqrt(var + LN_EPS)


def _mid_body(hl_ref, ha_ref, sgl_ref, sga_ref, nv_ref, mt_ref,
              g1_ref, c1_ref, g2_ref, c2_ref,
              w1_ref, bf1_ref, w2_ref, bf2_ref, m_ref, o_ref, sg_ref):
    i = pl.program_id(0)
    nv = nv_ref[...]
    m1, i1 = _gstats(sgl_ref[...], nv)
    m2, i2 = _gstats(sga_ref[...], nv)
    st = jnp.concatenate([m1, i1, m2, i2, jnp.zeros((GP, 4), jnp.float32)], 1)
    ns = _mm(mt_ref[...], st)
    hl = (hl_ref[...] - ns[:, 0:1]) * ns[:, 1:2] * g1_ref[...] + c1_ref[...]
    ha = (ha_ref[...] - ns[:, 2:3]) * ns[:, 3:4] * g2_ref[...] + c2_ref[...]
    op = hl + ha
    z = _mm(op, w1_ref[...]) + bf1_ref[...]
    ge = 0.5 * z * (1.0 + lax.erf(z * _SQ2I))
    out2 = _mm(ge, w2_ref[...]) + bf2_ref[...] + op
    o_ref[...] = out2
    _accum(i, sg_ref, _mm(m_ref[...], _stat_rows(out2)))


def _ln3_node(h_ref, o2_ref, sg_ref, nv_ref, mt_ref, g3_ref, c3_ref):
    m3, i3 = _gstats(sg_ref[...], nv_ref[...])
    st = jnp.concatenate([m3, i3, jnp.zeros((GP, 6), jnp.float32)], 1)
    ns = _mm(mt_ref[...], st)
    return (h_ref[...]
            + (o2_ref[...] - ns[:, 0:1]) * ns[:, 1:2] * g3_ref[...]
            + c3_ref[...])


def _finqkv_body(h_ref, o2_ref, sg_ref, nv_ref, mt_ref, g3_ref, c3_ref,
                 wq_ref, bq_ref, wk_ref, bk_ref, wv_ref, bv_ref,
                 o_ref, q_ref, k_ref, v_ref):
    hn = _ln3_node(h_ref, o2_ref, sg_ref, nv_ref, mt_ref, g3_ref, c3_ref)
    o_ref[...] = hn
    q_ref[...] = _mm(hn, wq_ref[...]) + bq_ref[...]
    k_ref[...] = _mm(hn, wk_ref[...]) + bk_ref[...]
    v_ref[...] = _mm(hn, wv_ref[...]) + bv_ref[...]


def _finpost_body(h_ref, o2_ref, sg_ref, nv_ref, mt_ref, g3_ref, c3_ref,
                  w1_ref, b1_ref, w2_ref, b2_ref, m_ref, o_ref, sgp_ref):
    i = pl.program_id(0)
    hn = _ln3_node(h_ref, o2_ref, sg_ref, nv_ref, mt_ref, g3_ref, c3_ref)
    o_ref[...] = hn
    z = jnp.maximum(_mm(hn, w1_ref[...]) + b1_ref[...], 0.0)
    hp = hn + _mm(z, w2_ref[...]) + b2_ref[...]
    _accum(i, sgp_ref, _mm(m_ref[...], hp))


def _read_body(sg_ref, dn_ref, w1_ref, b1_ref, w2_ref, b2_ref, o_ref):
    g = sg_ref[...] / dn_ref[...]
    z = jnp.maximum(_mm(g, w1_ref[...]) + b1_ref[...], 0.0)
    o_ref[...] = _mm(z, w2_ref[...]) + b2_ref[...]


_FULL2 = lambda i: (0, 0)
_TILE = lambda i: (i, 0)


def _spec_full(shape):
    return pl.BlockSpec(shape, _FULL2)


def _spec_tile(cols, rows=TN):
    return pl.BlockSpec((rows, cols), _TILE)


def kernel(x, edge_index, batch, lap_pe, params):
    batch = batch.astype(jnp.int32)
    epad = jnp.arange(EPAD, dtype=jnp.int32)
    src2d = jnp.concatenate(
        [edge_index[0].astype(jnp.int32), (epad * 997) % N]).reshape(NEB, EB)
    dst2d = jnp.concatenate(
        [edge_index[1].astype(jnp.int32), N + epad % (NP - N)]).reshape(NEB, EB)

    counts = jnp.bincount(batch, length=G)
    starts = jnp.cumsum(counts) - counts
    batch_pad = jnp.concatenate(
        [batch, jnp.full((NP - N,), PADG, jnp.int32)])
    counts_p = jnp.concatenate(
        [counts, jnp.array([NP - N], counts.dtype),
         jnp.zeros((GP - G - 1,), counts.dtype)])
    starts_p = jnp.concatenate(
        [starts, jnp.array([N], starts.dtype),
         jnp.full((GP - G - 1,), NP, starts.dtype)])
    ends_p = starts_p + counts_p
    gids = jnp.arange(GP, dtype=jnp.int32)
    mmat = (batch_pad[None, :] == gids[:, None]).astype(jnp.float32)
    mtmat = (batch_pad[:, None] == gids[None, :]).astype(jnp.float32)
    bq_col = batch_pad.reshape(NP, 1)
    bt = batch_pad.reshape(NT2, KT)
    first_g = batch_pad[0::T]
    last_g = batch_pad[T - 1::T]
    kv_lo = (starts_p[first_g] // KT).astype(jnp.int32).reshape(1, NT)
    kv_hi = ((ends_p[last_g] + (KT - 1)) // KT).astype(jnp.int32).reshape(1, NT)
    normv = jnp.maximum(counts_p.astype(jnp.float32) * H, 1.0).reshape(GP, 1)
    denom = jnp.maximum(counts_p.astype(jnp.float32), 1.0).reshape(GP, 1)
    zeros_np = jnp.zeros((NP, H), jnp.float32)
    xc = jnp.concatenate([x, lap_pe], axis=1)
    xc = jnp.concatenate(
        [xc, jnp.zeros((NP - N, D_IN + PE), jnp.float32)], axis=0)

    def lin(p):
        return p["w"], p["b"].reshape(1, -1)

    smem_spec = pl.BlockSpec(memory_space=pltpu.SMEM)

    f32 = jnp.float32

    def attnw(lp):
        return lin(lp["attn"]["q"]) + lin(lp["attn"]["k"]) + lin(lp["attn"]["v"])

    nm = params["node_mlp"]
    w1, b1 = lin(nm["l1"])
    w2, b2 = lin(nm["l2"])
    h, q, k, v = pl.pallas_call(
        _h0qkv_body,
        grid=(NTN,),
        in_specs=[_spec_tile(D_IN + PE), _spec_full((D_IN + PE, H)),
                  _spec_full((1, H)), _spec_full((H, H)), _spec_full((1, H))]
                 + [_spec_full((H, H)), _spec_full((1, H))] * 3,
        out_specs=[_spec_tile(H)] * 4,
        out_shape=[jax.ShapeDtypeStruct((NP, H), f32)] * 4,
    )(xc, w1, b1, w2, b2, *attnw(params["layers"][0]))

    for li, lp in enumerate(params["layers"]):
        agg2 = _sc_agg(h, src2d, dst2d, zeros_np)

        wo, bo = lin(lp["attn"]["o"])
        ha_pre, sg_ha = pl.pallas_call(
            _attn_body,
            grid=(NTN,),
            in_specs=[_spec_tile(H, T), _spec_full((NP, H)),
                      _spec_full((NP, H)),
                      pl.BlockSpec((T, 1), _TILE), _spec_full((NT2, KT)),
                      smem_spec, smem_spec,
                      _spec_full((H, H)), _spec_full((1, H)),
                      _spec_tile(H, T),
                      pl.BlockSpec((GP, T), lambda i: (0, i))],
            out_specs=[_spec_tile(H, T), pl.BlockSpec((GP, 8), _FULL2)],
            out_shape=[jax.ShapeDtypeStruct((NP, H), f32),
                       jax.ShapeDtypeStruct((GP, 8), f32)],
        )(q, k, v, bq_col, bt, kv_lo, kv_hi, wo, bo, h, mmat)

        gw1, gb1 = lin(lp["gin_mlp"]["l1"])
        gw2, gb2 = lin(lp["gin_mlp"]["l2"])
        geps = lp["gin_eps"].reshape(1, 1)
        hl_pre, sg_hl = pl.pallas_call(
            _gin_body,
            grid=(NTN,),
            in_specs=[_spec_tile(H),
                      pl.BlockSpec((TN, H), _TILE),
                      pl.BlockSpec((TN, H), lambda i: (i + NTN, 0)),
                      _spec_full((1, 1)),
                      _spec_full((H, H)), _spec_full((1, H)),
                      _spec_full((H, H)), _spec_full((1, H)),
                      pl.BlockSpec((GP, TN), lambda i: (0, i))],
            out_specs=[_spec_tile(H), pl.BlockSpec((GP, 8), _FULL2)],
            out_shape=[jax.ShapeDtypeStruct((NP, H), f32),
                       jax.ShapeDtypeStruct((GP, 8), f32)],
        )(h, agg2, agg2, geps, gw1, gb1, gw2, gb2, mmat)

        g1 = lp["norm1"]["g"].reshape(1, H)
        c1 = lp["norm1"]["b"].reshape(1, H)
        g2 = lp["norm2"]["g"].reshape(1, H)
        c2 = lp["norm2"]["b"].reshape(1, H)
        fw1, fb1 = lin(lp["ff"]["l1"])
        fw2, fb2 = lin(lp["ff"]["l2"])
        out2, sg_o = pl.pallas_call(
            _mid_body,
            grid=(NTN,),
            in_specs=[_spec_tile(H), _spec_tile(H), _spec_full((GP, 8)),
                      _spec_full((GP, 8)), _spec_full((GP, 1)),
                      _spec_tile(GP),
                      _spec_full((1, H)), _spec_full((1, H)),
                      _spec_full((1, H)), _spec_full((1, H)),
                      _spec_full((H, 2 * H)), _spec_full((1, 2 * H)),
                      _spec_full((2 * H, H)), _spec_full((1, H)),
                      pl.BlockSpec((GP, TN), lambda i: (0, i))],
            out_specs=[_spec_tile(H), pl.BlockSpec((GP, 8), _FULL2)],
            out_shape=[jax.ShapeDtypeStruct((NP, H), f32),
                       jax.ShapeDtypeStruct((GP, 8), f32)],
        )(hl_pre, ha_pre, sg_hl, sg_ha, normv, mtmat, g1, c1, g2, c2,
          fw1, fb1, fw2, fb2, mmat)

        g3 = lp["norm3"]["g"].reshape(1, H)
        c3 = lp["norm3"]["b"].reshape(1, H)
        if li < LAYERS - 1:
            h, q, k, v = pl.pallas_call(
                _finqkv_body,
                grid=(NTN,),
                in_specs=[_spec_tile(H), _spec_tile(H), _spec_full((GP, 8)),
                          _spec_full((GP, 1)), _spec_tile(GP),
                          _spec_full((1, H)), _spec_full((1, H))]
                         + [_spec_full((H, H)), _spec_full((1, H))] * 3,
                out_specs=[_spec_tile(H)] * 4,
                out_shape=[jax.ShapeDtypeStruct((NP, H), f32)] * 4,
            )(h, out2, sg_o, normv, mtmat, g3, c3,
              *attnw(params["layers"][li + 1]))
        else:
            pw1, pb1 = lin(params["postnet"]["l1"])
            pw2, pb2 = lin(params["postnet"]["l2"])
            _, sg_pool = pl.pallas_call(
                _finpost_body,
                grid=(NTN,),
                in_specs=[_spec_tile(H), _spec_tile(H), _spec_full((GP, 8)),
                          _spec_full((GP, 1)), _spec_tile(GP),
                          _spec_full((1, H)), _spec_full((1, H)),
                          _spec_full((H, H)), _spec_full((1, H)),
                          _spec_full((H, H)), _spec_full((1, H)),
                          pl.BlockSpec((GP, TN), lambda i: (0, i))],
                out_specs=[_spec_tile(H), pl.BlockSpec((GP, H), _FULL2)],
                out_shape=[jax.ShapeDtypeStruct((NP, H), f32),
                           jax.ShapeDtypeStruct((GP, H), f32)],
            )(h, out2, sg_o, normv, mtmat, g3, c3,
              pw1, pb1, pw2, pb2, mmat)

    rw1, rb1 = lin(params["readout"]["l1"])
    rw2, rb2 = lin(params["readout"]["l2"])
    out = pl.pallas_call(
        _read_body,
        grid=(1,),
        in_specs=[_spec_full((GP, H)), _spec_full((GP, 1)),
                  _spec_full((H, H)), _spec_full((1, H)),
                  _spec_full((H, 1)), _spec_full((1, 1))],
        out_specs=_spec_full((GP, 1)),
        out_shape=jax.ShapeDtypeStruct((GP, 1), jnp.float32),
    )(sg_pool, denom, rw1, rb1, rw2, rb2)
    return out[:G]
